# baseline (speedup 1.0000x reference)
import functools

import jax
import jax.numpy as jnp
from jax import lax
from jax.experimental import pallas as pl
from jax.experimental.pallas import tpu as pltpu
from jax.experimental.pallas import tpu_sc as plsc

EMB = 64
BATCH = 16384
N_SLOTS = 128
SLOT_W = 86400.0 / N_SLOTS

NC = 2
NS = 16
NW = NC * NS
BPW = BATCH // NW
CHUNK = 128
NCH = BPW // CHUNK
LANES = 16


def _body(cas_hbm, src_hbm, dst_hbm, times_hbm, ust_hbm, udt_hbm, cst_hbm,
          tt_hbm, out_hbm,
          sidx_v, didx_v, cidx_v, slot_v, times_v,
          src_v, dst_v, cas_v, ttc0_v, ttc1_v, gsem, osem):
  c = lax.axis_index("c")
  s = lax.axis_index("s")
  wid = s * NC + c
  base = wid * BPW

  pltpu.sync_copy(src_hbm.at[pl.ds(base, BPW)], sidx_v)
  pltpu.sync_copy(dst_hbm.at[pl.ds(base, BPW)], didx_v)
  pltpu.sync_copy(cas_hbm.at[pl.ds(base, BPW)], cidx_v)
  pltpu.sync_copy(times_hbm.at[pl.ds(base, BPW)], times_v)

  for i in range(BPW // LANES):
    t = times_v[pl.ds(i * LANES, LANES)]
    sl = jnp.clip((t / SLOT_W).astype(jnp.int32), 0, N_SLOTS - 1)
    slot_v[pl.ds(i * LANES, LANES)] = sl

  tt_bufs = [ttc0_v, ttc1_v]
  copies = []
  for j in range(NCH):
    sl_ = pl.ds(j * CHUNK, CHUNK)
    copies.append(pltpu.async_copy(ust_hbm.at[sidx_v.at[sl_]], src_v.at[sl_], gsem))
    copies.append(pltpu.async_copy(udt_hbm.at[didx_v.at[sl_]], dst_v.at[sl_], gsem))
    copies.append(pltpu.async_copy(cst_hbm.at[cidx_v.at[sl_]], cas_v.at[sl_], gsem))
  for j in (0, 1):
    sl_ = pl.ds(j * CHUNK, CHUNK)
    copies.append(pltpu.async_copy(tt_hbm.at[slot_v.at[sl_]], tt_bufs[j], gsem))
  for cp in copies:
    cp.wait()

  def add_tt(chunk, buf):
    @plsc.parallel_loop(0, CHUNK, 1, unroll=4)
    def _(r):
      for jj in range(EMB // LANES):
        o = jj * LANES
        cas_v[chunk * CHUNK + r, pl.ds(o, LANES)] = (
            cas_v[chunk * CHUNK + r, pl.ds(o, LANES)] + buf[r, pl.ds(o, LANES)])

  add_tt(0, ttc0_v)
  add_tt(1, ttc1_v)
  copies = []
  for j in (2, 3):
    sl_ = pl.ds(j * CHUNK, CHUNK)
    copies.append(pltpu.async_copy(tt_hbm.at[slot_v.at[sl_]], tt_bufs[j - 2], gsem))
  for cp in copies:
    cp.wait()
  add_tt(2, ttc0_v)
  add_tt(3, ttc1_v)

  rows = pl.ds(base, BPW)
  o1 = pltpu.async_copy(src_v, out_hbm.at[rows, pl.ds(0, EMB)], osem)
  o2 = pltpu.async_copy(dst_v, out_hbm.at[rows, pl.ds(EMB, EMB)], osem)
  o3 = pltpu.async_copy(cas_v, out_hbm.at[rows, pl.ds(2 * EMB, EMB)], osem)
  o1.wait()
  o2.wait()
  o3.wait()


@jax.jit
def kernel(cascades, src_idx, dst_idx, cas_pub_times, user_src_state,
           user_dst_state, cas_state, time_table):
  mesh = plsc.VectorSubcoreMesh(core_axis_name="c", subcore_axis_name="s")
  run = pl.kernel(
      _body,
      out_type=jax.ShapeDtypeStruct((BATCH, 3 * EMB), jnp.float32),
      mesh=mesh,
      compiler_params=pltpu.CompilerParams(use_tc_tiling_on_sc=False),
      scratch_types=[
          pltpu.VMEM((BPW,), jnp.int32),
          pltpu.VMEM((BPW,), jnp.int32),
          pltpu.VMEM((BPW,), jnp.int32),
          pltpu.VMEM((BPW,), jnp.int32),
          pltpu.VMEM((BPW,), jnp.float32),
          pltpu.VMEM((BPW, EMB), jnp.float32),
          pltpu.VMEM((BPW, EMB), jnp.float32),
          pltpu.VMEM((BPW, EMB), jnp.float32),
          pltpu.VMEM((CHUNK, EMB), jnp.float32),
          pltpu.VMEM((CHUNK, EMB), jnp.float32),
          pltpu.SemaphoreType.DMA,
          pltpu.SemaphoreType.DMA,
      ],
  )
  return run(cascades.astype(jnp.int32), src_idx, dst_idx, cas_pub_times,
             user_src_state, user_dst_state, cas_state, time_table)


# speedup vs baseline: 3.2195x; 3.2195x over previous
import functools

import jax
import jax.numpy as jnp
from jax import lax
from jax.experimental import pallas as pl
from jax.experimental.pallas import tpu as pltpu
from jax.experimental.pallas import tpu_sc as plsc

EMB = 64
PAD = 128
BATCH = 16384
N_SLOTS = 128
SLOT_W = 86400.0 / N_SLOTS

NC = 2
NS = 16
NW = NC * NS
BPW = BATCH // NW
CHUNK = 128
NCH = BPW // CHUNK
LANES = 16
RING = 8
LOOK = 6


def _i16(v):
  return jnp.full((LANES,), v, jnp.int32)


def _body(ss0, ss1, ss2, po0, po1, po2, slots_hbm, t0T, t1T, t2T, ttT,
          out0, out1, out2,
          ssv_v, slv_v, pos2_v, ring_v, out_v, tt_v, gsem):
  c = lax.axis_index("c")
  s = lax.axis_index("s")
  wid = s * NC + c
  base = wid * BPW

  iotas = [lax.iota(jnp.int32, LANES) + 16 * j for j in range(4)]

  def band(ss_hbm, pos_hbm, tT_hbm, out_hbm, with_tt):
    pltpu.sync_copy(ss_hbm.at[pl.ds(base, BPW)], ssv_v.at[pl.ds(0, BPW)])
    for q in range(NCH):
      pltpu.sync_copy(pos_hbm.at[pl.ds(base + q * CHUNK, CHUNK)],
                      pos2_v.at[q])
    if with_tt:
      pltpu.sync_copy(slots_hbm.at[pl.ds(base, BPW)],
                      slv_v.at[pl.ds(0, BPW)])

    def sid(r):
      return ssv_v[pl.ds(r, LANES)][0]

    def blk(r):
      return sid(r) // PAD

    def newblk(r):
      return jnp.where(r == 0, True, blk(r) != blk(jnp.maximum(r - 1, 0)))

    def fire(b, head):
      pltpu.async_copy(tT_hbm.at[:, pl.ds(b * PAD, PAD)],
                       ring_v.at[lax.rem(head, RING)], gsem)

    def drain():
      pltpu.make_async_copy(tT_hbm.at[:, pl.ds(0, PAD)], ring_v.at[0],
                            gsem).wait()

    def pro(r, head):
      @pl.when(newblk(r))
      def _():
        fire(blk(r), head)
      return head + newblk(r).astype(jnp.int32)

    head0 = lax.fori_loop(0, LOOK, pro, jnp.int32(0))

    def step(r, carry):
      head, cur = carry
      pf = jnp.logical_and(r + LOOK < BPW, newblk(r + LOOK))

      @pl.when(pf)
      def _():
        fire(blk(r + LOOK), head)

      head = head + pf.astype(jnp.int32)

      nb = newblk(r)

      @pl.when(nb)
      def _():
        drain()

      cur = cur + nb.astype(jnp.int32)
      slot = lax.rem(cur - 1, RING)

      col = lax.rem(sid(r), PAD)
      r2 = lax.rem(r, CHUNK)
      if with_tt:
        tcol = slv_v[pl.ds(r, LANES)][0]
      for j in range(4):
        val = plsc.load_gather(ring_v, [_i16(slot), iotas[j], _i16(col)])
        if with_tt:
          val = val + plsc.load_gather(tt_v, [iotas[j], _i16(tcol)])
        plsc.store_scatter(out_v, [_i16(r2), iotas[j]], val)

      @pl.when(r2 == CHUNK - 1)
      def _():
        q = r // CHUNK
        pltpu.sync_copy(out_v, out_hbm.at[pos2_v.at[q]])

      return head, cur

    lax.fori_loop(0, BPW, step, (head0, jnp.int32(0)))

  pltpu.sync_copy(ttT, tt_v)
  band(ss0, po0, t0T, out0, False)
  band(ss1, po1, t1T, out1, False)
  band(ss2, po2, t2T, out2, True)


@jax.jit
def kernel(cascades, src_idx, dst_idx, cas_pub_times, user_src_state,
           user_dst_state, cas_state, time_table):
  slot = jnp.clip((cas_pub_times / SLOT_W).astype(jnp.int32), 0, N_SLOTS - 1)
  cas32 = cascades.astype(jnp.int32)
  so = jnp.argsort(src_idx).astype(jnp.int32)
  do = jnp.argsort(dst_idx).astype(jnp.int32)
  co = jnp.argsort(cas32).astype(jnp.int32)
  ss = jnp.take(src_idx, so)
  ds_ = jnp.take(dst_idx, do)
  cs = jnp.take(cas32, co)
  slot_s = jnp.take(slot, co)

  mesh = plsc.VectorSubcoreMesh(core_axis_name="c", subcore_axis_name="s")
  run = pl.kernel(
      _body,
      out_type=[jax.ShapeDtypeStruct((BATCH, PAD), jnp.float32)] * 3,
      mesh=mesh,
      compiler_params=pltpu.CompilerParams(needs_layout_passes=False),
      scratch_types=[
          pltpu.VMEM((BPW + LANES,), jnp.int32),
          pltpu.VMEM((BPW + LANES,), jnp.int32),
          pltpu.VMEM((NCH, CHUNK), jnp.int32),
          pltpu.VMEM((RING, EMB, PAD), jnp.float32),
          pltpu.VMEM((CHUNK, PAD), jnp.float32),
          pltpu.VMEM((EMB, PAD), jnp.float32),
          pltpu.SemaphoreType.DMA,
      ],
  )
  sband, dband, cband = run(ss, ds_, cs, so, do, co, slot_s,
                            user_src_state.T, user_dst_state.T, cas_state.T,
                            time_table.T)
  return jnp.concatenate(
      [sband[:, :EMB], dband[:, :EMB], cband[:, :EMB]], axis=1)
